# baseline (speedup 1.0000x reference)
import functools

import jax
import jax.numpy as jnp
from jax import lax
from jax.experimental import pallas as pl
from jax.experimental.pallas import tpu as pltpu
from jax.experimental.pallas import tpu_sc as plsc

VOCAB = 1000000
EMBED = 32
NUM_CLASSES = 10
BATCH = 16384
SEQ = 200

L = 16
NC = 2
NS = 16
NW = NC * NS
CHUNK = 16
G = CHUNK * SEQ
GSUB = 128
NGSUB = G // GSUB
CPW = BATCH // (CHUNK * NW)

PACK = 8
N2 = PACK * L
SLAB = VOCAB // PACK
VBLK = 5000


def _transform_body(emb_ref, w_ref, bvec_ref, out_ref):
    out_ref[...] = (
        jnp.dot(emb_ref[...], w_ref[...], preferred_element_type=jnp.float32)
        + bvec_ref[...]
    )


_transform = pl.pallas_call(
    _transform_body,
    grid=(SLAB // VBLK,),
    in_specs=[
        pl.BlockSpec((VBLK, PACK * EMBED), lambda i: (i, 0)),
        pl.BlockSpec((PACK * EMBED, N2), lambda i: (0, 0)),
        pl.BlockSpec((1, N2), lambda i: (0, 0)),
    ],
    out_specs=pl.BlockSpec((VBLK, N2), lambda i: (i, 0)),
    out_shape=jax.ShapeDtypeStruct((SLAB, N2), jnp.float32),
)


@functools.partial(
    pl.kernel,
    out_type=jax.ShapeDtypeStruct((BATCH, L), jnp.float32),
    mesh=plsc.VectorSubcoreMesh(core_axis_name="c", subcore_axis_name="s"),
    scratch_types=[
        pltpu.VMEM((CHUNK, SEQ), jnp.int32),
        pltpu.VMEM((CHUNK, SEQ), jnp.int32),
        pltpu.VMEM((G, L), jnp.float32),
        pltpu.VMEM((G, L), jnp.float32),
        pltpu.VMEM((CHUNK, L), jnp.float32),
        pltpu.SemaphoreType.DMA,
        pltpu.SemaphoreType.DMA,
    ],
    compiler_params=pltpu.CompilerParams(use_tc_tiling_on_sc=False),
)
def _pool(x_hbm, t_hbm, out_hbm, idx_a, idx_b,
          rows_a, rows_b, acc_v, sem_a, sem_b):
    wid = lax.axis_index("s") * NC + lax.axis_index("c")
    base = wid * CPW

    def issue(bw, idx_v, rows_v, sem):
        pltpu.sync_copy(x_hbm.at[pl.ds(bw * CHUNK, CHUNK)], idx_v)
        for r in range(CHUNK):
            pltpu.async_copy(
                t_hbm.at[idx_v.at[r, pl.ds(0, GSUB)]],
                rows_v.at[pl.ds(r * SEQ, GSUB)],
                sem,
            )
            pltpu.async_copy(
                t_hbm.at[idx_v.at[r, pl.ds(GSUB, SEQ - GSUB)]],
                rows_v.at[pl.ds(r * SEQ + GSUB, SEQ - GSUB)],
                sem,
            )

    def drain(rows_v, sem):
        pltpu.make_async_copy(t_hbm.at[pl.ds(0, G)], rows_v, sem).wait()

    def consume(bw, rows_v):
        zero = jnp.zeros((L,), jnp.float32)

        def body(s, accs):
            return tuple(accs[r] + rows_v[r * SEQ + s] for r in range(CHUNK))

        accs = lax.fori_loop(0, SEQ, body, (zero,) * CHUNK)
        for r in range(CHUNK):
            acc_v[r] = accs[r]
        pltpu.sync_copy(acc_v, out_hbm.at[pl.ds(bw * CHUNK, CHUNK)])

    issue(base, idx_a, rows_a, sem_a)

    def outer(i, carry):
        bw_a = base + 2 * i
        bw_b = bw_a + 1
        issue(bw_b, idx_b, rows_b, sem_b)
        drain(rows_a, sem_a)
        consume(bw_a, rows_a)
        issue(jnp.minimum(bw_a + 2, base + CPW - 1), idx_a, rows_a, sem_a)
        drain(rows_b, sem_b)
        consume(bw_b, rows_b)
        return carry

    lax.fori_loop(0, CPW // 2, outer, 0)
    drain(rows_a, sem_a)


def kernel(x, emb_table, W, b):
    inv_s = jnp.float32(1.0 / SEQ)
    w_pad = jnp.zeros((EMBED, L), jnp.float32).at[:, :NUM_CLASSES].set(W) * inv_s
    b_pad = jnp.zeros((L,), jnp.float32).at[:NUM_CLASSES].set(b) * inv_s
    b_big = jnp.tile(b_pad, PACK)[None, :]
    w_big = jnp.kron(jnp.eye(PACK, dtype=jnp.float32), w_pad)
    t = _transform(emb_table.reshape(SLAB, PACK * EMBED), w_big, b_big)
    out = _pool(x.astype(jnp.int32), t.reshape(VOCAB, L))
    return out[:, :NUM_CLASSES]


# speedup vs baseline: 1.3440x; 1.3440x over previous
import functools

import jax
import jax.numpy as jnp
from jax import lax
from jax.experimental import pallas as pl
from jax.experimental.pallas import tpu as pltpu
from jax.experimental.pallas import tpu_sc as plsc

VOCAB = 1000000
EMBED = 32
NUM_CLASSES = 10
BATCH = 16384
SEQ = 200

L = 16
NC = 2
NS = 16
NW = NC * NS
CHUNK = 16
G = CHUNK * SEQ
GSUB = 128
NGSUB = G // GSUB
CPW = BATCH // (CHUNK * NW)

PACK = 8
N2 = PACK * L
SLAB = VOCAB // PACK
VBLK = 1000


def _transform_body(emb_ref, w_ref, bvec_ref, out_ref):
    e3 = emb_ref[...]
    e256 = jnp.concatenate([e3[j] for j in range(PACK)], axis=1)
    out_ref[...] = (
        jnp.dot(e256, w_ref[...], preferred_element_type=jnp.float32)
        + bvec_ref[...]
    )


_transform = pl.pallas_call(
    _transform_body,
    grid=(SLAB // VBLK,),
    in_specs=[
        pl.BlockSpec((PACK, VBLK, EMBED), lambda i: (0, i, 0)),
        pl.BlockSpec((PACK * EMBED, N2), lambda i: (0, 0)),
        pl.BlockSpec((1, N2), lambda i: (0, 0)),
    ],
    out_specs=pl.BlockSpec((VBLK, N2), lambda i: (i, 0)),
    out_shape=jax.ShapeDtypeStruct((SLAB, N2), jnp.float32),
)


@functools.partial(
    pl.kernel,
    out_type=jax.ShapeDtypeStruct((BATCH, L), jnp.float32),
    mesh=plsc.VectorSubcoreMesh(core_axis_name="c", subcore_axis_name="s"),
    scratch_types=[
        pltpu.VMEM((CHUNK, SEQ), jnp.int32),
        pltpu.VMEM((CHUNK, SEQ), jnp.int32),
        pltpu.VMEM((CHUNK, SEQ), jnp.int32),
        pltpu.VMEM((CHUNK, SEQ), jnp.int32),
        pltpu.VMEM((G, L), jnp.float32),
        pltpu.VMEM((G, L), jnp.float32),
        pltpu.VMEM((CHUNK, L), jnp.float32),
        pltpu.SemaphoreType.DMA,
        pltpu.SemaphoreType.DMA,
    ],
    compiler_params=pltpu.CompilerParams(use_tc_tiling_on_sc=False),
)
def _pool(x_hbm, t_hbm, out_hbm, idx_a, idx_b, tix_a, tix_b,
          rows_a, rows_b, acc_v, sem_a, sem_b):
    wid = lax.axis_index("s") * NC + lax.axis_index("c")
    base = wid * CPW

    def issue(bw, idx_v, tix_v, rows_v, sem):
        pltpu.sync_copy(x_hbm.at[pl.ds(bw * CHUNK, CHUNK)], idx_v)
        for r in range(CHUNK):
            for q in range(13):
                o = min(q * L, SEQ - L)
                v = idx_v[r, pl.ds(o, L)]
                j0 = lax.shift_right_logical(v, 17)
                r0 = v - j0 * SLAB
                c = lax.shift_right_arithmetic(r0 - SLAB, 31) + 1
                tix_v[r, pl.ds(o, L)] = (r0 - c * SLAB) * PACK + j0 + c
        for r in range(CHUNK):
            pltpu.async_copy(
                t_hbm.at[tix_v.at[r, pl.ds(0, GSUB)]],
                rows_v.at[pl.ds(r * SEQ, GSUB)],
                sem,
            )
            pltpu.async_copy(
                t_hbm.at[tix_v.at[r, pl.ds(GSUB, SEQ - GSUB)]],
                rows_v.at[pl.ds(r * SEQ + GSUB, SEQ - GSUB)],
                sem,
            )

    def drain(rows_v, sem):
        pltpu.make_async_copy(t_hbm.at[pl.ds(0, G)], rows_v, sem).wait()

    def consume(bw, rows_v):
        zero = jnp.zeros((L,), jnp.float32)

        def body(s, accs):
            return tuple(accs[r] + rows_v[r * SEQ + s] for r in range(CHUNK))

        accs = lax.fori_loop(0, SEQ, body, (zero,) * CHUNK)
        for r in range(CHUNK):
            acc_v[r] = accs[r]
        pltpu.sync_copy(acc_v, out_hbm.at[pl.ds(bw * CHUNK, CHUNK)])

    issue(base, idx_a, tix_a, rows_a, sem_a)

    def outer(i, carry):
        bw_a = base + 2 * i
        bw_b = bw_a + 1
        issue(bw_b, idx_b, tix_b, rows_b, sem_b)
        drain(rows_a, sem_a)
        consume(bw_a, rows_a)
        issue(jnp.minimum(bw_a + 2, base + CPW - 1), idx_a, tix_a, rows_a,
              sem_a)
        drain(rows_b, sem_b)
        consume(bw_b, rows_b)
        return carry

    lax.fori_loop(0, CPW // 2, outer, 0)
    drain(rows_a, sem_a)


def kernel(x, emb_table, W, b):
    inv_s = jnp.float32(1.0 / SEQ)
    w_pad = jnp.zeros((EMBED, L), jnp.float32).at[:, :NUM_CLASSES].set(W) * inv_s
    b_pad = jnp.zeros((L,), jnp.float32).at[:NUM_CLASSES].set(b) * inv_s
    b_big = jnp.tile(b_pad, PACK)[None, :]
    w_big = jnp.kron(jnp.eye(PACK, dtype=jnp.float32), w_pad)
    t = _transform(emb_table.reshape(PACK, SLAB, EMBED), w_big, b_big)
    out = _pool(x.astype(jnp.int32), t.reshape(VOCAB, L))
    return out[:, :NUM_CLASSES]


# speedup vs baseline: 1.4555x; 1.0830x over previous
import functools

import jax
import jax.numpy as jnp
from jax import lax
from jax.experimental import pallas as pl
from jax.experimental.pallas import tpu as pltpu
from jax.experimental.pallas import tpu_sc as plsc

VOCAB = 1000000
EMBED = 32
NUM_CLASSES = 10
BATCH = 16384
SEQ = 200

L = 16
NC = 2
NS = 16
NW = NC * NS
CHUNK = 16
G = CHUNK * SEQ
GSUB = 128
NGSUB = G // GSUB
CPW = BATCH // (CHUNK * NW)

PACK = 8
N2 = PACK * L
SLAB = VOCAB // PACK
VBLK = 5000


def _transform_body(emb_ref, w_ref, bvec_ref, out_ref):
    e3 = emb_ref[...]
    e256 = jnp.concatenate([e3[j] for j in range(PACK)], axis=1)
    out_ref[...] = (
        jnp.dot(e256, w_ref[...], preferred_element_type=jnp.float32)
        + bvec_ref[...]
    )


_transform = pl.pallas_call(
    _transform_body,
    grid=(SLAB // VBLK,),
    in_specs=[
        pl.BlockSpec((PACK, VBLK, EMBED), lambda i: (0, i, 0)),
        pl.BlockSpec((PACK * EMBED, N2), lambda i: (0, 0)),
        pl.BlockSpec((1, N2), lambda i: (0, 0)),
    ],
    out_specs=pl.BlockSpec((VBLK, N2), lambda i: (i, 0)),
    out_shape=jax.ShapeDtypeStruct((SLAB, N2), jnp.float32),
)


XBLK = 1024


def _remap_body(x_ref, out_ref):
    v = x_ref[...]
    j0 = lax.shift_right_logical(v, 17)
    r0 = v - j0 * SLAB
    c = lax.shift_right_arithmetic(r0 - SLAB, 31) + 1
    out_ref[...] = (r0 - c * SLAB) * PACK + j0 + c


_remap = pl.pallas_call(
    _remap_body,
    grid=(BATCH // XBLK,),
    in_specs=[pl.BlockSpec((XBLK, SEQ), lambda i: (i, 0))],
    out_specs=pl.BlockSpec((XBLK, SEQ), lambda i: (i, 0)),
    out_shape=jax.ShapeDtypeStruct((BATCH, SEQ), jnp.int32),
)


@functools.partial(
    pl.kernel,
    out_type=jax.ShapeDtypeStruct((BATCH, L), jnp.float32),
    mesh=plsc.VectorSubcoreMesh(core_axis_name="c", subcore_axis_name="s"),
    scratch_types=[
        pltpu.VMEM((CHUNK, SEQ), jnp.int32),
        pltpu.VMEM((CHUNK, SEQ), jnp.int32),
        pltpu.VMEM((G, L), jnp.float32),
        pltpu.VMEM((G, L), jnp.float32),
        pltpu.VMEM((CHUNK, L), jnp.float32),
        pltpu.SemaphoreType.DMA,
        pltpu.SemaphoreType.DMA,
    ],
    compiler_params=pltpu.CompilerParams(use_tc_tiling_on_sc=False),
)
def _pool(x_hbm, t_hbm, out_hbm, idx_a, idx_b,
          rows_a, rows_b, acc_v, sem_a, sem_b):
    wid = lax.axis_index("s") * NC + lax.axis_index("c")
    base = wid * CPW

    def issue(bw, idx_v, rows_v, sem):
        pltpu.sync_copy(x_hbm.at[pl.ds(bw * CHUNK, CHUNK)], idx_v)
        for r in range(CHUNK):
            pltpu.async_copy(
                t_hbm.at[idx_v.at[r, pl.ds(0, GSUB)]],
                rows_v.at[pl.ds(r * SEQ, GSUB)],
                sem,
            )
            pltpu.async_copy(
                t_hbm.at[idx_v.at[r, pl.ds(GSUB, SEQ - GSUB)]],
                rows_v.at[pl.ds(r * SEQ + GSUB, SEQ - GSUB)],
                sem,
            )

    def drain(rows_v, sem):
        pltpu.make_async_copy(t_hbm.at[pl.ds(0, G)], rows_v, sem).wait()

    def consume(bw, rows_v):
        zero = jnp.zeros((L,), jnp.float32)

        def body(s, accs):
            return tuple(accs[r] + rows_v[r * SEQ + s] for r in range(CHUNK))

        accs = lax.fori_loop(0, SEQ, body, (zero,) * CHUNK)
        for r in range(CHUNK):
            acc_v[r] = accs[r]
        pltpu.sync_copy(acc_v, out_hbm.at[pl.ds(bw * CHUNK, CHUNK)])

    issue(base, idx_a, rows_a, sem_a)

    def outer(i, carry):
        bw_a = base + 2 * i
        bw_b = bw_a + 1
        issue(bw_b, idx_b, rows_b, sem_b)
        drain(rows_a, sem_a)
        consume(bw_a, rows_a)
        issue(jnp.minimum(bw_a + 2, base + CPW - 1), idx_a, rows_a, sem_a)
        drain(rows_b, sem_b)
        consume(bw_b, rows_b)
        return carry

    lax.fori_loop(0, CPW // 2, outer, 0)
    drain(rows_a, sem_a)


def kernel(x, emb_table, W, b):
    inv_s = jnp.float32(1.0 / SEQ)
    w_pad = jnp.zeros((EMBED, L), jnp.float32).at[:, :NUM_CLASSES].set(W) * inv_s
    b_pad = jnp.zeros((L,), jnp.float32).at[:NUM_CLASSES].set(b) * inv_s
    b_big = jnp.tile(b_pad, PACK)[None, :]
    w_big = jnp.kron(jnp.eye(PACK, dtype=jnp.float32), w_pad)
    t = _transform(emb_table.reshape(PACK, SLAB, EMBED), w_big, b_big)
    out = _pool(_remap(x.astype(jnp.int32)), t.reshape(VOCAB, L))
    return out[:, :NUM_CLASSES]
